# baseline (speedup 1.0000x reference)
import functools

import jax
import jax.numpy as jnp
from jax import lax
from jax.experimental import pallas as pl
from jax.experimental.pallas import tpu as pltpu
from jax.experimental.pallas import tpu_sc as plsc

N = 50000
NP = 50176
E = 800000
EP = 819200
F_IN = 100
EMB = 128
K = 30
G = 31
OUT = 235
DENSE = 1408

_RT = 784
_ACC = 800
_SENT = 792
_QCAP = 256
_EROWS = EP // 128

f32 = jnp.float32
i32 = jnp.int32


def _mesh():
    return plsc.VectorSubcoreMesh(
        core_axis_name="c", subcore_axis_name="s", num_cores=2, num_subcores=16)


def _iota16():
    return lax.broadcasted_iota(i32, (16,), 0)


def _sc_deg(dst2d, zeros_h, ones_h):
    @functools.partial(
        pl.kernel,
        out_type=jax.ShapeDtypeStruct((NP, EMB), f32),
        compiler_params=pltpu.CompilerParams(needs_layout_passes=False),
        mesh=_mesh(),
        scratch_types=[
            pltpu.VMEM_SHARED((12672, EMB), f32),
            pltpu.VMEM((8, 128), i32),
            pltpu.VMEM((8, 128), i32),
            pltpu.VMEM((128, EMB), f32),
        ],
    )
    def k(dst_h, z_h, ones_h_, out_h, acc, dstbuf, offbuf, onesbuf):
        c = lax.axis_index("c")
        s = lax.axis_index("s")
        pltpu.sync_copy(ones_h_, onesbuf)
        for p in range(2):
            q = p * 2 + c
            base = q * 12544
            pltpu.sync_copy(z_h, acc.at[pl.ds(s * 792, 792)])
            plsc.subcore_barrier()

            def chunk(i, _):
                ro = s * 400 + i * 8
                pltpu.sync_copy(dst_h.at[pl.ds(ro, 8)], dstbuf)
                for r in range(8):
                    for kk in range(8):
                        d = dstbuf[r, pl.ds(kk * 16, 16)]
                        off = d - base
                        ok = (d >= base) & (d < base + 12544)
                        offbuf[r, pl.ds(kk * 16, 16)] = jnp.where(ok, off, 12544)
                for rj in range(8):
                    pltpu.sync_copy(onesbuf, acc.at[offbuf.at[rj]], add=True)
                return 0

            lax.fori_loop(0, 50, chunk, 0)
            plsc.subcore_barrier()
            pltpu.sync_copy(acc.at[pl.ds(s * 784, 784)],
                            out_h.at[pl.ds(base + s * 784, 784)])
            plsc.subcore_barrier()

    return k(dst2d, zeros_h, ones_h)


def _sc_build(src2d, dst2d, dinv1d):
    @functools.partial(
        pl.kernel,
        out_type=(
            jax.ShapeDtypeStruct((64, _QCAP, 1, 128), i32),
            jax.ShapeDtypeStruct((64, _QCAP, 1, 128), f32),
            jax.ShapeDtypeStruct((64, 1, 16), i32),
        ),
        compiler_params=pltpu.CompilerParams(needs_layout_passes=False),
        mesh=_mesh(),
        scratch_types=[
            pltpu.VMEM((NP,), f32),
            pltpu.VMEM((8, 128), i32),
            pltpu.VMEM((8, 128), i32),
            pltpu.VMEM((1280,), i32),
            pltpu.VMEM((1280,), f32),
            pltpu.VMEM((1280,), i32),
            pltpu.VMEM((1280,), f32),
            pltpu.VMEM((1, 128), i32),
            pltpu.VMEM((1, 128), f32),
            pltpu.VMEM((1, 16), i32),
        ],
    )
    def k(src_h, dst_h, dinv_h, qpk_h, qnm_h, qsz_h,
          dinv_v, srcbuf, dstbuf, q0p, q0n, q1p, q1n, stp, stn, szb):
        c = lax.axis_index("c")
        s = lax.axis_index("s")
        w = c * 16 + s
        pltpu.sync_copy(dinv_h, dinv_v)
        it16 = _iota16()

        queues = [(q0p, q0n, w), (q1p, q1n, 32 + w)]

        def flush(qp, qn, rid, fc, nf):
            def fbody(j, fcj):
                for g in range(8):
                    stp[0, pl.ds(g * 16, 16)] = qp[pl.ds(j * 128 + g * 16, 16)]
                    stn[0, pl.ds(g * 16, 16)] = qn[pl.ds(j * 128 + g * 16, 16)]
                pltpu.sync_copy(stp, qpk_h.at[rid, fcj])
                pltpu.sync_copy(stn, qnm_h.at[rid, fcj])
                return fcj + 1

            fc = lax.fori_loop(0, nf, fbody, fc)
            for g in range(8):
                pv = qp[pl.ds(nf * 128 + g * 16, 16)]
                nv = qn[pl.ds(nf * 128 + g * 16, 16)]
                qp[pl.ds(g * 16, 16)] = pv
                qn[pl.ds(g * 16, 16)] = nv
            return fc

        def chunk(i, carry):
            qt0, fc0, qt1, fc1 = carry
            ro = i * 8
            pltpu.sync_copy(src_h.at[pl.ds(ro, 8)], srcbuf)
            pltpu.sync_copy(dst_h.at[pl.ds(ro, 8)], dstbuf)
            for r in range(8):
                for kk in range(8):
                    svec = srcbuf[r, pl.ds(kk * 16, 16)]
                    dvec = dstbuf[r, pl.ds(kk * 16, 16)]
                    dsg = plsc.load_gather(dinv_v, [svec])
                    ddg = plsc.load_gather(dinv_v, [dvec])
                    nm = dsg * ddg
                    for (qp, qn, rid), which in ((queues[0], 0), (queues[1], 1)):
                        base = rid * _RT
                        m = (dvec >= base) & (dvec < base + _RT)
                        pk = svec * 1024 + (dvec - base)
                        qt = qt0 if which == 0 else qt1
                        plsc.store_compressed(qp.at[pl.ds(qt, 16)], pk, mask=m)
                        plsc.store_compressed(qn.at[pl.ds(qt, 16)], nm, mask=m)
                        cnt = jnp.sum(m.astype(i32))
                        if which == 0:
                            qt0 = qt + cnt
                        else:
                            qt1 = qt + cnt
            nf0 = qt0 // 128
            fc0 = flush(q0p, q0n, w, fc0, nf0)
            qt0 = qt0 - nf0 * 128
            nf1 = qt1 // 128
            fc1 = flush(q1p, q1n, 32 + w, fc1, nf1)
            qt1 = qt1 - nf1 * 128
            return qt0, fc0, qt1, fc1

        qt0, fc0, qt1, fc1 = lax.fori_loop(
            0, 800, chunk, (jnp.int32(0), jnp.int32(0), jnp.int32(0), jnp.int32(0)))

        sent_pk = jnp.full((16,), _SENT, i32)
        sent_nm = jnp.zeros((16,), f32)
        for (qp, qn, rid), qt, fc in ((queues[0], qt0, fc0), (queues[1], qt1, fc1)):
            for g in range(8):
                qp[pl.ds(qt + g * 16, 16)] = sent_pk
                qn[pl.ds(qt + g * 16, 16)] = sent_nm
            for g in range(8):
                stp[0, pl.ds(g * 16, 16)] = qp[pl.ds(g * 16, 16)]
                stn[0, pl.ds(g * 16, 16)] = qn[pl.ds(g * 16, 16)]
            pltpu.sync_copy(stp, qpk_h.at[rid, fc])
            pltpu.sync_copy(stn, qnm_h.at[rid, fc])
            total = fc + jnp.where(qt > 0, 1, 0).astype(i32)
            szb[0, pl.ds(0, 16)] = jnp.broadcast_to(total, (16,))
            pltpu.sync_copy(szb, qsz_h.at[rid])

    return k(src2d, dst2d, dinv1d)


def _sc_segsum(qpk, qnm, qsz, tab, dinv1d, zeros_h):
    @functools.partial(
        pl.kernel,
        out_type=jax.ShapeDtypeStruct((NP, EMB), f32),
        compiler_params=pltpu.CompilerParams(needs_layout_passes=False),
        mesh=_mesh(),
        scratch_types=[
            pltpu.VMEM((_ACC, EMB), f32),
            pltpu.VMEM((128, EMB), f32),
            pltpu.VMEM((1, 128), i32),
            pltpu.VMEM((1, 128), f32),
            pltpu.VMEM((128,), i32),
            pltpu.VMEM((_RT,), f32),
            pltpu.VMEM((1, 16), i32),
            pltpu.SemaphoreType.DMA,
        ],
    )
    def k(qpk_h, qnm_h, qsz_h, tab_h, dinv_h, z_h, out_h,
          acc, rowbuf, pkbuf, nmbuf, gidxv, dv_v, szv, sem):
        c = lax.axis_index("c")
        s = lax.axis_index("s")
        w = c * 16 + s
        it16 = _iota16()
        for p in range(2):
            rid = p * 32 + w
            base = rid * _RT
            pltpu.sync_copy(z_h, acc)
            pltpu.sync_copy(qsz_h.at[rid], szv)
            nch = jnp.max(szv[0, pl.ds(0, 16)])

            def chunk(j, _):
                pltpu.sync_copy(qpk_h.at[rid, j], pkbuf)
                pltpu.sync_copy(qnm_h.at[rid, j], nmbuf)
                for g in range(8):
                    pk = pkbuf[0, pl.ds(g * 16, 16)]
                    gidxv[pl.ds(g * 16, 16)] = pk // 1024
                pltpu.async_copy(tab_h.at[gidxv], rowbuf, sem).wait()
                offs = []
                nms = []
                for g in range(8):
                    pk = pkbuf[0, pl.ds(g * 16, 16)]
                    offs.append(pk - (pk // 1024) * 1024)
                    nms.append(nmbuf[0, pl.ds(g * 16, 16)])

                def colbody(col, _2):
                    colv = jnp.broadcast_to(col, (16,))
                    for g in range(8):
                        lanes = g * 16 + it16
                        v = plsc.load_gather(rowbuf, [lanes, colv])
                        t = v * nms[g]
                        plsc.addupdate_scatter(acc, [offs[g], colv], t)
                    return 0

                lax.fori_loop(0, 128, colbody, 0)
                return 0

            lax.fori_loop(0, nch, chunk, 0)

            pltpu.sync_copy(dinv_h.at[pl.ds(base, _RT)], dv_v)
            for cc in range(7):
                pltpu.sync_copy(tab_h.at[pl.ds(base + cc * 112, 112)],
                                rowbuf.at[pl.ds(0, 112)])
                for g in range(7):
                    rowoff = cc * 112 + g * 16
                    lanes = g * 16 + it16
                    dv = dv_v[pl.ds(rowoff, 16)]
                    nn = dv * dv
                    offv = rowoff + it16

                    def slbody(col, _2):
                        colv = jnp.broadcast_to(col, (16,))
                        v = plsc.load_gather(rowbuf, [lanes, colv])
                        t = v * nn
                        plsc.addupdate_scatter(acc, [offv, colv], t)
                        return 0

                    lax.fori_loop(0, 128, slbody, 0)

            pltpu.sync_copy(acc.at[pl.ds(0, _RT)],
                            out_h.at[pl.ds(base, _RT)])

    return k(qpk, qnm, qsz, tab, dinv1d, zeros_h)


def _sc_gather(idx, x1, x2, x3, x4f):
    @functools.partial(
        pl.kernel,
        out_type=tuple(
            jax.ShapeDtypeStruct((1024, EMB), f32) for _ in range(4)),
        compiler_params=pltpu.CompilerParams(needs_layout_passes=False),
        mesh=_mesh(),
        scratch_types=[
            pltpu.VMEM((1, 32), i32),
            pltpu.VMEM((32, EMB), f32),
            pltpu.VMEM((32, EMB), f32),
            pltpu.VMEM((32, EMB), f32),
            pltpu.VMEM((32, EMB), f32),
            pltpu.SemaphoreType.DMA,
        ],
    )
    def k(idx_h, x1_h, x2_h, x3_h, x4_h, p1_h, p2_h, p3_h, p4_h,
          idxv, r1, r2, r3, r4, sem):
        c = lax.axis_index("c")
        s = lax.axis_index("s")
        w = c * 16 + s
        pltpu.sync_copy(idx_h.at[w], idxv)
        ds_ = [pltpu.async_copy(x1_h.at[idxv.at[0]], r1, sem),
               pltpu.async_copy(x2_h.at[idxv.at[0]], r2, sem),
               pltpu.async_copy(x3_h.at[idxv.at[0]], r3, sem),
               pltpu.async_copy(x4_h.at[idxv.at[0]], r4, sem)]
        for d in ds_:
            d.wait()
        pltpu.sync_copy(r1, p1_h.at[pl.ds(w * 32, 32)])
        pltpu.sync_copy(r2, p2_h.at[pl.ds(w * 32, 32)])
        pltpu.sync_copy(r3, p3_h.at[pl.ds(w * 32, 32)])
        pltpu.sync_copy(r4, p4_h.at[pl.ds(w * 32, 32)])

    return k(idx, x1, x2, x3, x4f)


_BN = 1024


def _tc_mm0(x, W1, deg):
    def body(x_ref, w_ref, dg_ref, dinv_ref, hw_ref):
        d = dg_ref[:, 0:1] + 1.0
        dinv_ref[...] = lax.rsqrt(d)
        hw_ref[...] = jnp.dot(x_ref[...], w_ref[...],
                              preferred_element_type=f32)

    return pl.pallas_call(
        body,
        grid=(NP // _BN,),
        in_specs=[
            pl.BlockSpec((_BN, F_IN), lambda i: (i, 0)),
            pl.BlockSpec((F_IN, EMB), lambda i: (0, 0)),
            pl.BlockSpec((_BN, EMB), lambda i: (i, 0)),
        ],
        out_specs=[
            pl.BlockSpec((_BN, 1), lambda i: (i, 0)),
            pl.BlockSpec((_BN, EMB), lambda i: (i, 0)),
        ],
        out_shape=[
            jax.ShapeDtypeStruct((NP, 1), f32),
            jax.ShapeDtypeStruct((NP, EMB), f32),
        ],
    )(x, W1, deg)


def _tc_layer(sa, b, Wn):
    def body(s_ref, b_ref, w_ref, x_ref, hwn_ref):
        t = jnp.tanh(s_ref[...] + b_ref[...])
        x_ref[...] = t
        hwn_ref[...] = jnp.dot(t, w_ref[...], preferred_element_type=f32)

    return pl.pallas_call(
        body,
        grid=(NP // _BN,),
        in_specs=[
            pl.BlockSpec((_BN, EMB), lambda i: (i, 0)),
            pl.BlockSpec((1, EMB), lambda i: (0, 0)),
            pl.BlockSpec((EMB, EMB), lambda i: (0, 0)),
        ],
        out_specs=[
            pl.BlockSpec((_BN, EMB), lambda i: (i, 0)),
            pl.BlockSpec((_BN, EMB), lambda i: (i, 0)),
        ],
        out_shape=[
            jax.ShapeDtypeStruct((NP, EMB), f32),
            jax.ShapeDtypeStruct((NP, EMB), f32),
        ],
    )(sa, b, Wn)


def _tc_layer4(s4, b4):
    def body(s_ref, b_ref, x_ref):
        x_ref[...] = jnp.tanh(s_ref[...] + b_ref[...])

    return pl.pallas_call(
        body,
        grid=(NP // _BN,),
        in_specs=[
            pl.BlockSpec((_BN, EMB), lambda i: (i, 0)),
            pl.BlockSpec((1, EMB), lambda i: (0, 0)),
        ],
        out_specs=pl.BlockSpec((_BN, EMB), lambda i: (i, 0)),
        out_shape=jax.ShapeDtypeStruct((NP, EMB), f32),
    )(s4, b4)


_NROW = 392


def _tc_topk(score2d, batch2d):
    def body(score_ref, batch_ref, idx_ref, maskc_ref):
        idx_ref[...] = jnp.zeros((32, 1, 32), i32)
        maskc_ref[...] = jnp.zeros((1024, 1), f32)
        score = score_ref[...]
        batchv = batch_ref[...]
        iota2 = (lax.broadcasted_iota(i32, (_NROW, 128), 0) * 128
                 + lax.broadcasted_iota(i32, (_NROW, 128), 1))
        lane = lax.broadcasted_iota(i32, (1, 32), 1)

        def gbody(g, _):
            sg0 = jnp.where(batchv == g, score, -jnp.inf)

            def kbody(kk, carry):
                sg, idxrow = carry
                v = jnp.max(sg)
                flat = jnp.where(sg == v, iota2, jnp.int32(2 ** 30))
                fi = jnp.min(flat)
                sg = jnp.where(iota2 == fi, -jnp.inf, sg)
                idxrow = jnp.where(lane == kk, jnp.minimum(fi, N - 1), idxrow)
                valid = (v > -jnp.inf).astype(f32)
                maskc_ref[pl.ds(g * 32 + kk, 1), :] = valid.reshape(1, 1)
                return sg, idxrow

            sg, idxrow = lax.fori_loop(0, K, kbody,
                                       (sg0, jnp.zeros((1, 32), i32)))
            idx_ref[pl.ds(g, 1)] = idxrow.reshape(1, 1, 32)
            return 0

        lax.fori_loop(0, G, gbody, 0)

    return pl.pallas_call(
        body,
        out_shape=[
            jax.ShapeDtypeStruct((32, 1, 32), i32),
            jax.ShapeDtypeStruct((1024, 1), f32),
        ],
    )(score2d, batch2d)


def _tc_head_a(p1, p2, p3, p4, maskc, W5p, b5, W6r, b6):
    def body(p1_ref, p2_ref, p3_ref, p4_ref, m_ref, w5_ref, b5_ref,
             w6_ref, b6_ref, y6_ref):
        m = m_ref[...]
        xc = jnp.concatenate(
            [p1_ref[...], p2_ref[...], p3_ref[...], p4_ref[:, :16]],
            axis=-1) * m
        y5 = jax.nn.relu(
            jnp.dot(xc, w5_ref[...], preferred_element_type=f32) + b5_ref[...])
        zp = jnp.max(y5.reshape(512, 2, 64), axis=1)
        zp3 = zp.reshape(32, 16, 64)
        u = jnp.concatenate([zp3[:, j:j + 11, :] for j in range(5)], axis=-1)
        y6 = jax.nn.relu(
            jnp.dot(u.reshape(352, 320), w6_ref[...],
                    preferred_element_type=f32) + b6_ref[...])
        y6_ref[...] = y6.reshape(32, 11, EMB)

    return pl.pallas_call(
        body,
        out_shape=jax.ShapeDtypeStruct((32, 11, EMB), f32),
    )(p1, p2, p3, p4, maskc, W5p, b5, W6r, b6)


def _tc_head_b(emb2, W1c, b1, W2c, b2):
    def body(e_ref, w1_ref, b1_ref, w2_ref, b2_ref, o_ref, acc_ref):
        i = pl.program_id(0)

        @pl.when(i == 0)
        def _():
            acc_ref[...] = jnp.zeros((1, 384), f32)

        acc_ref[...] += jnp.dot(e_ref[...], w1_ref[...],
                                preferred_element_type=f32)

        @pl.when(i == G - 1)
        def _():
            h = jnp.tanh(acc_ref[...] + b1_ref[...])
            o_ref[...] = jnp.tanh(
                jnp.dot(h, w2_ref[...], preferred_element_type=f32)
                + b2_ref[...])

    return pl.pallas_call(
        body,
        grid=(G,),
        in_specs=[
            pl.BlockSpec((1, DENSE), lambda i: (0, i)),
            pl.BlockSpec((DENSE, 384), lambda i: (i, 0)),
            pl.BlockSpec((1, 384), lambda i: (0, 0)),
            pl.BlockSpec((384, OUT), lambda i: (0, 0)),
            pl.BlockSpec((1, OUT), lambda i: (0, 0)),
        ],
        out_specs=pl.BlockSpec((1, OUT), lambda i: (0, 0)),
        out_shape=jax.ShapeDtypeStruct((1, OUT), f32),
        scratch_shapes=[pltpu.VMEM((1, 384), f32)],
    )(emb2, W1c, b1, W2c, b2)


def kernel(x, edge_index, batch, W1g, b1g, W2g, b2g, W3g, b3g, W4g, b4g,
           conv5_w, conv5_b, conv6_w, conv6_b, cls1_W, cls1_b, cls2_W, cls2_b):
    xpad = jnp.pad(x, ((0, NP - N), (0, 0)))
    epad = EP - E
    src2d = jnp.concatenate(
        [edge_index[0], jnp.zeros((epad,), i32)]).reshape(_EROWS, 128)
    dst2d = jnp.concatenate(
        [edge_index[1], jnp.full((epad,), N, i32)]).reshape(_EROWS, 128)
    zeros_deg = jnp.zeros((792, EMB), f32)
    zeros_acc = jnp.zeros((_ACC, EMB), f32)
    ones_h = jnp.ones((128, EMB), f32)

    deg = _sc_deg(dst2d, zeros_deg, ones_h)
    dinv, hw1 = _tc_mm0(xpad, W1g, deg)
    dinv1d = dinv.reshape(NP)

    qpk, qnm, qsz = _sc_build(src2d, dst2d, dinv1d)

    s1 = _sc_segsum(qpk, qnm, qsz, hw1, dinv1d, zeros_acc)
    x1, hw2 = _tc_layer(s1, b1g.reshape(1, EMB), W2g)

    s2 = _sc_segsum(qpk, qnm, qsz, hw2, dinv1d, zeros_acc)
    x2, hw3 = _tc_layer(s2, b2g.reshape(1, EMB), W3g)

    s3 = _sc_segsum(qpk, qnm, qsz, hw3, dinv1d, zeros_acc)
    W4p = jnp.pad(W4g, ((0, 0), (0, EMB - 1)))
    x3, hw4 = _tc_layer(s3, b3g.reshape(1, EMB), W4p)

    s4 = _sc_segsum(qpk, qnm, qsz, hw4, dinv1d, zeros_acc)
    b4r = jnp.broadcast_to(b4g.reshape(1, 1), (1, EMB))
    x4f = _tc_layer4(s4, b4r)

    sflat = x4f[:N, 0]
    pad = NP - N
    score2d = jnp.concatenate(
        [sflat, jnp.full((pad,), -jnp.inf, f32)]).reshape(_NROW, 128)
    batch2d = jnp.concatenate(
        [batch, jnp.full((pad,), -1, i32)]).reshape(_NROW, 128)
    idx, maskc = _tc_topk(score2d, batch2d)

    p1, p2, p3, p4 = _sc_gather(idx, x1, x2, x3, x4f)

    W5p = jnp.pad(conv5_w.reshape(EMB // 2, 385).T, ((0, 15), (0, 0)))
    W6r = jnp.transpose(conv6_w, (2, 1, 0)).reshape(320, EMB)
    y6 = _tc_head_a(p1, p2, p3, p4, maskc, W5p, conv5_b.reshape(1, 64),
                    W6r, conv6_b.reshape(1, EMB))

    emb2 = jnp.transpose(y6[:G], (0, 2, 1)).reshape(1, G * DENSE)
    out = _tc_head_b(emb2, cls1_W, cls1_b.reshape(1, 384),
                     cls2_W, cls2_b.reshape(1, OUT))
    return out[0]


# speedup vs baseline: 2.3134x; 2.3134x over previous
import functools

import jax
import jax.numpy as jnp
from jax import lax
from jax.experimental import pallas as pl
from jax.experimental.pallas import tpu as pltpu
from jax.experimental.pallas import tpu_sc as plsc

N = 50000
NP = 50176
E = 800000
EP = 819200
F_IN = 100
EMB = 128
K = 30
G = 31
OUT = 235
DENSE = 1408

_RT = 784
_ACC = 800
_SENT = 792
_QCAP = 256
_EROWS = EP // 128

f32 = jnp.float32
i32 = jnp.int32


def _mesh():
    return plsc.VectorSubcoreMesh(
        core_axis_name="c", subcore_axis_name="s", num_cores=2, num_subcores=16)


def _iota16():
    return lax.broadcasted_iota(i32, (16,), 0)


def _sc_deg(dst2d, zeros_h, ones_h):
    @functools.partial(
        pl.kernel,
        out_type=jax.ShapeDtypeStruct((NP, EMB), f32),
        compiler_params=pltpu.CompilerParams(needs_layout_passes=False),
        mesh=_mesh(),
        scratch_types=[
            pltpu.VMEM_SHARED((12672, EMB), f32),
            pltpu.VMEM((8, 128), i32),
            pltpu.VMEM((8, 128), i32),
            pltpu.VMEM((128, EMB), f32),
        ],
    )
    def k(dst_h, z_h, ones_h_, out_h, acc, dstbuf, offbuf, onesbuf):
        c = lax.axis_index("c")
        s = lax.axis_index("s")
        pltpu.sync_copy(ones_h_, onesbuf)
        for p in range(2):
            q = p * 2 + c
            base = q * 12544
            pltpu.sync_copy(z_h, acc.at[pl.ds(s * 792, 792)])
            plsc.subcore_barrier()

            def chunk(i, _):
                ro = s * 400 + i * 8
                pltpu.sync_copy(dst_h.at[pl.ds(ro, 8)], dstbuf)
                for r in range(8):
                    for kk in range(8):
                        d = dstbuf[r, pl.ds(kk * 16, 16)]
                        off = d - base
                        ok = (d >= base) & (d < base + 12544)
                        offbuf[r, pl.ds(kk * 16, 16)] = jnp.where(ok, off, 12544)
                for rj in range(8):
                    pltpu.sync_copy(onesbuf, acc.at[offbuf.at[rj]], add=True)
                return 0

            lax.fori_loop(0, 50, chunk, 0)
            plsc.subcore_barrier()
            pltpu.sync_copy(acc.at[pl.ds(s * 784, 784)],
                            out_h.at[pl.ds(base + s * 784, 784)])
            plsc.subcore_barrier()

    return k(dst2d, zeros_h, ones_h)


def _sc_build(src2d, dst2d, dinv1d):
    @functools.partial(
        pl.kernel,
        out_type=(
            jax.ShapeDtypeStruct((64, _QCAP, 1, 128), i32),
            jax.ShapeDtypeStruct((64, _QCAP, 1, 128), f32),
            jax.ShapeDtypeStruct((64, 1, 16), i32),
        ),
        compiler_params=pltpu.CompilerParams(needs_layout_passes=False),
        mesh=_mesh(),
        scratch_types=[
            pltpu.VMEM((NP,), f32),
            pltpu.VMEM((8, 128), i32),
            pltpu.VMEM((8, 128), i32),
            pltpu.VMEM((1280,), i32),
            pltpu.VMEM((1280,), f32),
            pltpu.VMEM((1280,), i32),
            pltpu.VMEM((1280,), f32),
            pltpu.VMEM((1, 128), i32),
            pltpu.VMEM((1, 128), f32),
            pltpu.VMEM((1, 16), i32),
        ],
    )
    def k(src_h, dst_h, dinv_h, qpk_h, qnm_h, qsz_h,
          dinv_v, srcbuf, dstbuf, q0p, q0n, q1p, q1n, stp, stn, szb):
        c = lax.axis_index("c")
        s = lax.axis_index("s")
        w = c * 16 + s
        pltpu.sync_copy(dinv_h, dinv_v)
        it16 = _iota16()

        queues = [(q0p, q0n, w), (q1p, q1n, 32 + w)]

        def flush(qp, qn, rid, fc, nf):
            def fbody(j, fcj):
                for g in range(8):
                    stp[0, pl.ds(g * 16, 16)] = qp[pl.ds(j * 128 + g * 16, 16)]
                    stn[0, pl.ds(g * 16, 16)] = qn[pl.ds(j * 128 + g * 16, 16)]
                pltpu.sync_copy(stp, qpk_h.at[rid, fcj])
                pltpu.sync_copy(stn, qnm_h.at[rid, fcj])
                return fcj + 1

            fc = lax.fori_loop(0, nf, fbody, fc)
            for g in range(8):
                pv = qp[pl.ds(nf * 128 + g * 16, 16)]
                nv = qn[pl.ds(nf * 128 + g * 16, 16)]
                qp[pl.ds(g * 16, 16)] = pv
                qn[pl.ds(g * 16, 16)] = nv
            return fc

        def chunk(i, carry):
            qt0, fc0, qt1, fc1 = carry
            ro = i * 8
            pltpu.sync_copy(src_h.at[pl.ds(ro, 8)], srcbuf)
            pltpu.sync_copy(dst_h.at[pl.ds(ro, 8)], dstbuf)
            for r in range(8):
                for kk in range(8):
                    svec = srcbuf[r, pl.ds(kk * 16, 16)]
                    dvec = dstbuf[r, pl.ds(kk * 16, 16)]
                    dsg = plsc.load_gather(dinv_v, [svec])
                    ddg = plsc.load_gather(dinv_v, [dvec])
                    nm = dsg * ddg
                    for (qp, qn, rid), which in ((queues[0], 0), (queues[1], 1)):
                        base = rid * _RT
                        m = (dvec >= base) & (dvec < base + _RT)
                        pk = svec * 1024 + (dvec - base)
                        qt = qt0 if which == 0 else qt1
                        plsc.store_compressed(qp.at[pl.ds(qt, 16)], pk, mask=m)
                        plsc.store_compressed(qn.at[pl.ds(qt, 16)], nm, mask=m)
                        cnt = jnp.sum(m.astype(i32))
                        if which == 0:
                            qt0 = qt + cnt
                        else:
                            qt1 = qt + cnt
            nf0 = qt0 // 128
            fc0 = flush(q0p, q0n, w, fc0, nf0)
            qt0 = qt0 - nf0 * 128
            nf1 = qt1 // 128
            fc1 = flush(q1p, q1n, 32 + w, fc1, nf1)
            qt1 = qt1 - nf1 * 128
            return qt0, fc0, qt1, fc1

        qt0, fc0, qt1, fc1 = lax.fori_loop(
            0, 800, chunk, (jnp.int32(0), jnp.int32(0), jnp.int32(0), jnp.int32(0)))

        sent_pk = jnp.full((16,), _SENT, i32)
        sent_nm = jnp.zeros((16,), f32)
        for (qp, qn, rid), qt, fc in ((queues[0], qt0, fc0), (queues[1], qt1, fc1)):
            for g in range(8):
                qp[pl.ds(qt + g * 16, 16)] = sent_pk
                qn[pl.ds(qt + g * 16, 16)] = sent_nm
            for g in range(8):
                stp[0, pl.ds(g * 16, 16)] = qp[pl.ds(g * 16, 16)]
                stn[0, pl.ds(g * 16, 16)] = qn[pl.ds(g * 16, 16)]
            pltpu.sync_copy(stp, qpk_h.at[rid, fc])
            pltpu.sync_copy(stn, qnm_h.at[rid, fc])
            total = fc + jnp.where(qt > 0, 1, 0).astype(i32)
            szb[0, pl.ds(0, 16)] = jnp.broadcast_to(total, (16,))
            pltpu.sync_copy(szb, qsz_h.at[rid])

    return k(src2d, dst2d, dinv1d)


def _sc_segsum(qpk, qnm, qsz, tab, dinv1d, zeros_h):
    @functools.partial(
        pl.kernel,
        out_type=jax.ShapeDtypeStruct((NP, EMB), f32),
        compiler_params=pltpu.CompilerParams(needs_layout_passes=False),
        mesh=_mesh(),
        scratch_types=[
            pltpu.VMEM((_ACC, EMB), f32),
            pltpu.VMEM((128, EMB), f32),
            pltpu.VMEM((1, 128), i32),
            pltpu.VMEM((1, 128), f32),
            pltpu.VMEM((128,), i32),
            pltpu.VMEM((_RT,), f32),
            pltpu.VMEM((1, 16), i32),
            pltpu.SemaphoreType.DMA,
        ],
    )
    def k(qpk_h, qnm_h, qsz_h, tab_h, dinv_h, z_h, out_h,
          acc, rowbuf, pkbuf, nmbuf, gidxv, dv_v, szv, sem):
        c = lax.axis_index("c")
        s = lax.axis_index("s")
        w = c * 16 + s
        it16 = _iota16()
        for p in range(2):
            rid = p * 32 + w
            base = rid * _RT
            pltpu.sync_copy(z_h, acc)
            pltpu.sync_copy(qsz_h.at[rid], szv)
            nch = jnp.max(szv[0, pl.ds(0, 16)])

            def chunk(j, _):
                pltpu.sync_copy(qpk_h.at[rid, j], pkbuf)
                pltpu.sync_copy(qnm_h.at[rid, j], nmbuf)
                for g in range(8):
                    pk = pkbuf[0, pl.ds(g * 16, 16)]
                    gidxv[pl.ds(g * 16, 16)] = pk // 1024
                pltpu.async_copy(tab_h.at[gidxv], rowbuf, sem).wait()
                offs = []
                nms = []
                for g in range(8):
                    pk = pkbuf[0, pl.ds(g * 16, 16)]
                    offs.append(pk - (pk // 1024) * 1024)
                    nms.append(nmbuf[0, pl.ds(g * 16, 16)])

                for g in range(8):
                    lanes = g * 16 + it16
                    off16 = offs[g]
                    nm16 = nms[g]

                    def gsweep(t, _2):
                        cvec = t - it16
                        m = (cvec >= 0) & (cvec < 128)
                        cc = jnp.bitwise_and(cvec, 127)
                        v = plsc.load_gather(rowbuf, [lanes, cc], mask=m)
                        tt = v * nm16
                        plsc.addupdate_scatter(acc, [off16, cc], tt, mask=m)
                        return 0

                    lax.fori_loop(0, 143, gsweep, 0)
                return 0

            lax.fori_loop(0, nch, chunk, 0)

            pltpu.sync_copy(dinv_h.at[pl.ds(base, _RT)], dv_v)
            for cc in range(7):
                pltpu.sync_copy(tab_h.at[pl.ds(base + cc * 112, 112)],
                                rowbuf.at[pl.ds(0, 112)])
                for g in range(7):
                    rowoff = cc * 112 + g * 16
                    lanes = g * 16 + it16
                    dv = dv_v[pl.ds(rowoff, 16)]
                    nn = dv * dv
                    offv = rowoff + it16

                    def slbody(col, _2):
                        colv = jnp.bitwise_and(col + it16, 127)
                        v = plsc.load_gather(rowbuf, [lanes, colv])
                        t = v * nn
                        plsc.addupdate_scatter(acc, [offv, colv], t)
                        return 0

                    lax.fori_loop(0, 128, slbody, 0)

            pltpu.sync_copy(acc.at[pl.ds(0, _RT)],
                            out_h.at[pl.ds(base, _RT)])

    return k(qpk, qnm, qsz, tab, dinv1d, zeros_h)


def _sc_gather(idx, x1, x2, x3, x4f):
    @functools.partial(
        pl.kernel,
        out_type=tuple(
            jax.ShapeDtypeStruct((1024, EMB), f32) for _ in range(4)),
        compiler_params=pltpu.CompilerParams(needs_layout_passes=False),
        mesh=_mesh(),
        scratch_types=[
            pltpu.VMEM((1, 32), i32),
            pltpu.VMEM((32, EMB), f32),
            pltpu.VMEM((32, EMB), f32),
            pltpu.VMEM((32, EMB), f32),
            pltpu.VMEM((32, EMB), f32),
            pltpu.SemaphoreType.DMA,
        ],
    )
    def k(idx_h, x1_h, x2_h, x3_h, x4_h, p1_h, p2_h, p3_h, p4_h,
          idxv, r1, r2, r3, r4, sem):
        c = lax.axis_index("c")
        s = lax.axis_index("s")
        w = c * 16 + s
        pltpu.sync_copy(idx_h.at[w], idxv)
        ds_ = [pltpu.async_copy(x1_h.at[idxv.at[0]], r1, sem),
               pltpu.async_copy(x2_h.at[idxv.at[0]], r2, sem),
               pltpu.async_copy(x3_h.at[idxv.at[0]], r3, sem),
               pltpu.async_copy(x4_h.at[idxv.at[0]], r4, sem)]
        for d in ds_:
            d.wait()
        pltpu.sync_copy(r1, p1_h.at[pl.ds(w * 32, 32)])
        pltpu.sync_copy(r2, p2_h.at[pl.ds(w * 32, 32)])
        pltpu.sync_copy(r3, p3_h.at[pl.ds(w * 32, 32)])
        pltpu.sync_copy(r4, p4_h.at[pl.ds(w * 32, 32)])

    return k(idx, x1, x2, x3, x4f)


_BN = 1024


def _tc_mm0(x, W1, deg):
    def body(x_ref, w_ref, dg_ref, dinv_ref, hw_ref):
        d = dg_ref[:, 0:1] + 1.0
        dinv_ref[...] = lax.rsqrt(d)
        hw_ref[...] = jnp.dot(x_ref[...], w_ref[...],
                              preferred_element_type=f32)

    return pl.pallas_call(
        body,
        grid=(NP // _BN,),
        in_specs=[
            pl.BlockSpec((_BN, F_IN), lambda i: (i, 0)),
            pl.BlockSpec((F_IN, EMB), lambda i: (0, 0)),
            pl.BlockSpec((_BN, EMB), lambda i: (i, 0)),
        ],
        out_specs=[
            pl.BlockSpec((_BN, 1), lambda i: (i, 0)),
            pl.BlockSpec((_BN, EMB), lambda i: (i, 0)),
        ],
        out_shape=[
            jax.ShapeDtypeStruct((NP, 1), f32),
            jax.ShapeDtypeStruct((NP, EMB), f32),
        ],
    )(x, W1, deg)


def _tc_layer(sa, b, Wn):
    def body(s_ref, b_ref, w_ref, x_ref, hwn_ref):
        t = jnp.tanh(s_ref[...] + b_ref[...])
        x_ref[...] = t
        hwn_ref[...] = jnp.dot(t, w_ref[...], preferred_element_type=f32)

    return pl.pallas_call(
        body,
        grid=(NP // _BN,),
        in_specs=[
            pl.BlockSpec((_BN, EMB), lambda i: (i, 0)),
            pl.BlockSpec((1, EMB), lambda i: (0, 0)),
            pl.BlockSpec((EMB, EMB), lambda i: (0, 0)),
        ],
        out_specs=[
            pl.BlockSpec((_BN, EMB), lambda i: (i, 0)),
            pl.BlockSpec((_BN, EMB), lambda i: (i, 0)),
        ],
        out_shape=[
            jax.ShapeDtypeStruct((NP, EMB), f32),
            jax.ShapeDtypeStruct((NP, EMB), f32),
        ],
    )(sa, b, Wn)


def _tc_layer4(s4, b4):
    def body(s_ref, b_ref, x_ref):
        x_ref[...] = jnp.tanh(s_ref[...] + b_ref[...])

    return pl.pallas_call(
        body,
        grid=(NP // _BN,),
        in_specs=[
            pl.BlockSpec((_BN, EMB), lambda i: (i, 0)),
            pl.BlockSpec((1, EMB), lambda i: (0, 0)),
        ],
        out_specs=pl.BlockSpec((_BN, EMB), lambda i: (i, 0)),
        out_shape=jax.ShapeDtypeStruct((NP, EMB), f32),
    )(s4, b4)


_NROW = 392


def _tc_topk(score2d, batch2d):
    def body(score_ref, batch_ref, idx_ref, maskc_ref):
        idx_ref[...] = jnp.zeros((32, 1, 32), i32)
        maskc_ref[...] = jnp.zeros((1024, 1), f32)
        score = score_ref[...]
        batchv = batch_ref[...]
        iota2 = (lax.broadcasted_iota(i32, (_NROW, 128), 0) * 128
                 + lax.broadcasted_iota(i32, (_NROW, 128), 1))
        lane = lax.broadcasted_iota(i32, (1, 32), 1)

        def gbody(g, _):
            sg0 = jnp.where(batchv == g, score, -jnp.inf)

            def kbody(kk, carry):
                sg, idxrow = carry
                v = jnp.max(sg)
                flat = jnp.where(sg == v, iota2, jnp.int32(2 ** 30))
                fi = jnp.min(flat)
                sg = jnp.where(iota2 == fi, -jnp.inf, sg)
                idxrow = jnp.where(lane == kk, jnp.minimum(fi, N - 1), idxrow)
                valid = (v > -jnp.inf).astype(f32)
                maskc_ref[pl.ds(g * 32 + kk, 1), :] = valid.reshape(1, 1)
                return sg, idxrow

            sg, idxrow = lax.fori_loop(0, K, kbody,
                                       (sg0, jnp.zeros((1, 32), i32)))
            idx_ref[pl.ds(g, 1)] = idxrow.reshape(1, 1, 32)
            return 0

        lax.fori_loop(0, G, gbody, 0)

    return pl.pallas_call(
        body,
        out_shape=[
            jax.ShapeDtypeStruct((32, 1, 32), i32),
            jax.ShapeDtypeStruct((1024, 1), f32),
        ],
    )(score2d, batch2d)


def _tc_head_a(p1, p2, p3, p4, maskc, W5p, b5, W6r, b6):
    def body(p1_ref, p2_ref, p3_ref, p4_ref, m_ref, w5_ref, b5_ref,
             w6_ref, b6_ref, y6_ref):
        m = m_ref[...]
        xc = jnp.concatenate(
            [p1_ref[...], p2_ref[...], p3_ref[...], p4_ref[:, :16]],
            axis=-1) * m
        y5 = jax.nn.relu(
            jnp.dot(xc, w5_ref[...], preferred_element_type=f32) + b5_ref[...])
        zp = jnp.max(y5.reshape(512, 2, 64), axis=1)
        zp3 = zp.reshape(32, 16, 64)
        u = jnp.concatenate([zp3[:, j:j + 11, :] for j in range(5)], axis=-1)
        y6 = jax.nn.relu(
            jnp.dot(u.reshape(352, 320), w6_ref[...],
                    preferred_element_type=f32) + b6_ref[...])
        y6_ref[...] = y6.reshape(32, 11, EMB)

    return pl.pallas_call(
        body,
        out_shape=jax.ShapeDtypeStruct((32, 11, EMB), f32),
    )(p1, p2, p3, p4, maskc, W5p, b5, W6r, b6)


def _tc_head_b(emb2, W1c, b1, W2c, b2):
    def body(e_ref, w1_ref, b1_ref, w2_ref, b2_ref, o_ref, acc_ref):
        i = pl.program_id(0)

        @pl.when(i == 0)
        def _():
            acc_ref[...] = jnp.zeros((1, 384), f32)

        acc_ref[...] += jnp.dot(e_ref[...], w1_ref[...],
                                preferred_element_type=f32)

        @pl.when(i == G - 1)
        def _():
            h = jnp.tanh(acc_ref[...] + b1_ref[...])
            o_ref[...] = jnp.tanh(
                jnp.dot(h, w2_ref[...], preferred_element_type=f32)
                + b2_ref[...])

    return pl.pallas_call(
        body,
        grid=(G,),
        in_specs=[
            pl.BlockSpec((1, DENSE), lambda i: (0, i)),
            pl.BlockSpec((DENSE, 384), lambda i: (i, 0)),
            pl.BlockSpec((1, 384), lambda i: (0, 0)),
            pl.BlockSpec((384, OUT), lambda i: (0, 0)),
            pl.BlockSpec((1, OUT), lambda i: (0, 0)),
        ],
        out_specs=pl.BlockSpec((1, OUT), lambda i: (0, 0)),
        out_shape=jax.ShapeDtypeStruct((1, OUT), f32),
        scratch_shapes=[pltpu.VMEM((1, 384), f32)],
    )(emb2, W1c, b1, W2c, b2)


def kernel(x, edge_index, batch, W1g, b1g, W2g, b2g, W3g, b3g, W4g, b4g,
           conv5_w, conv5_b, conv6_w, conv6_b, cls1_W, cls1_b, cls2_W, cls2_b):
    xpad = jnp.pad(x, ((0, NP - N), (0, 0)))
    epad = EP - E
    src2d = jnp.concatenate(
        [edge_index[0], jnp.zeros((epad,), i32)]).reshape(_EROWS, 128)
    dst2d = jnp.concatenate(
        [edge_index[1], jnp.full((epad,), N, i32)]).reshape(_EROWS, 128)
    zeros_deg = jnp.zeros((792, EMB), f32)
    zeros_acc = jnp.zeros((_ACC, EMB), f32)
    ones_h = jnp.ones((128, EMB), f32)

    deg = _sc_deg(dst2d, zeros_deg, ones_h)
    dinv, hw1 = _tc_mm0(xpad, W1g, deg)
    dinv1d = dinv.reshape(NP)

    qpk, qnm, qsz = _sc_build(src2d, dst2d, dinv1d)

    s1 = _sc_segsum(qpk, qnm, qsz, hw1, dinv1d, zeros_acc)
    x1, hw2 = _tc_layer(s1, b1g.reshape(1, EMB), W2g)

    s2 = _sc_segsum(qpk, qnm, qsz, hw2, dinv1d, zeros_acc)
    x2, hw3 = _tc_layer(s2, b2g.reshape(1, EMB), W3g)

    s3 = _sc_segsum(qpk, qnm, qsz, hw3, dinv1d, zeros_acc)
    W4p = jnp.pad(W4g, ((0, 0), (0, EMB - 1)))
    x3, hw4 = _tc_layer(s3, b3g.reshape(1, EMB), W4p)

    s4 = _sc_segsum(qpk, qnm, qsz, hw4, dinv1d, zeros_acc)
    b4r = jnp.broadcast_to(b4g.reshape(1, 1), (1, EMB))
    x4f = _tc_layer4(s4, b4r)

    sflat = x4f[:N, 0]
    pad = NP - N
    score2d = jnp.concatenate(
        [sflat, jnp.full((pad,), -jnp.inf, f32)]).reshape(_NROW, 128)
    batch2d = jnp.concatenate(
        [batch, jnp.full((pad,), -1, i32)]).reshape(_NROW, 128)
    idx, maskc = _tc_topk(score2d, batch2d)

    p1, p2, p3, p4 = _sc_gather(idx, x1, x2, x3, x4f)

    W5p = jnp.pad(conv5_w.reshape(EMB // 2, 385).T, ((0, 15), (0, 0)))
    W6r = jnp.transpose(conv6_w, (2, 1, 0)).reshape(320, EMB)
    y6 = _tc_head_a(p1, p2, p3, p4, maskc, W5p, conv5_b.reshape(1, 64),
                    W6r, conv6_b.reshape(1, EMB))

    emb2 = jnp.transpose(y6[:G], (0, 2, 1)).reshape(1, G * DENSE)
    out = _tc_head_b(emb2, cls1_W, cls1_b.reshape(1, 384),
                     cls2_W, cls2_b.reshape(1, OUT))
    return out[0]


# speedup vs baseline: 3.8996x; 1.6857x over previous
import functools

import jax
import jax.numpy as jnp
from jax import lax
from jax.experimental import pallas as pl
from jax.experimental.pallas import tpu as pltpu
from jax.experimental.pallas import tpu_sc as plsc

N = 50000
NP = 50176
E = 800000
EP = 819200
F_IN = 100
EMB = 128
K = 30
G = 31
OUT = 235
DENSE = 1408

_RT = 784
_ACC = 800
_SENT = 792
_QCAP = 256
_EROWS = EP // 128

f32 = jnp.float32
i32 = jnp.int32


def _mesh():
    return plsc.VectorSubcoreMesh(
        core_axis_name="c", subcore_axis_name="s", num_cores=2, num_subcores=16)


def _iota16():
    return lax.broadcasted_iota(i32, (16,), 0)


def _sc_deg(dst2d, zeros_h, ones_h):
    @functools.partial(
        pl.kernel,
        out_type=jax.ShapeDtypeStruct((NP, EMB), f32),
        compiler_params=pltpu.CompilerParams(needs_layout_passes=False),
        mesh=_mesh(),
        scratch_types=[
            pltpu.VMEM_SHARED((12672, EMB), f32),
            pltpu.VMEM((8, 128), i32),
            pltpu.VMEM((8, 128), i32),
            pltpu.VMEM((128, EMB), f32),
        ],
    )
    def k(dst_h, z_h, ones_h_, out_h, acc, dstbuf, offbuf, onesbuf):
        c = lax.axis_index("c")
        s = lax.axis_index("s")
        pltpu.sync_copy(ones_h_, onesbuf)
        for p in range(2):
            q = p * 2 + c
            base = q * 12544
            pltpu.sync_copy(z_h, acc.at[pl.ds(s * 792, 792)])
            plsc.subcore_barrier()

            def chunk(i, _):
                ro = s * 400 + i * 8
                pltpu.sync_copy(dst_h.at[pl.ds(ro, 8)], dstbuf)
                for r in range(8):
                    for kk in range(8):
                        d = dstbuf[r, pl.ds(kk * 16, 16)]
                        off = d - base
                        ok = (d >= base) & (d < base + 12544)
                        offbuf[r, pl.ds(kk * 16, 16)] = jnp.where(ok, off, 12544)
                for rj in range(8):
                    pltpu.sync_copy(onesbuf, acc.at[offbuf.at[rj]], add=True)
                return 0

            lax.fori_loop(0, 50, chunk, 0)
            plsc.subcore_barrier()
            pltpu.sync_copy(acc.at[pl.ds(s * 784, 784)],
                            out_h.at[pl.ds(base + s * 784, 784)])
            plsc.subcore_barrier()

    return k(dst2d, zeros_h, ones_h)


def _sc_build(src2d, dst2d, dinv1d):
    @functools.partial(
        pl.kernel,
        out_type=(
            jax.ShapeDtypeStruct((64, _QCAP, 1, 128), i32),
            jax.ShapeDtypeStruct((64, _QCAP, 1, 128), f32),
            jax.ShapeDtypeStruct((64, 1, 16), i32),
        ),
        compiler_params=pltpu.CompilerParams(needs_layout_passes=False),
        mesh=_mesh(),
        scratch_types=[
            pltpu.VMEM((NP,), f32),
            pltpu.VMEM((8, 128), i32),
            pltpu.VMEM((8, 128), i32),
            pltpu.VMEM((1280,), i32),
            pltpu.VMEM((1280,), f32),
            pltpu.VMEM((1280,), i32),
            pltpu.VMEM((1280,), f32),
            pltpu.VMEM((1, 128), i32),
            pltpu.VMEM((1, 128), f32),
            pltpu.VMEM((1, 16), i32),
        ],
    )
    def k(src_h, dst_h, dinv_h, qpk_h, qnm_h, qsz_h,
          dinv_v, srcbuf, dstbuf, q0p, q0n, q1p, q1n, stp, stn, szb):
        c = lax.axis_index("c")
        s = lax.axis_index("s")
        w = c * 16 + s
        pltpu.sync_copy(dinv_h, dinv_v)
        it16 = _iota16()

        queues = [(q0p, q0n, w), (q1p, q1n, 32 + w)]

        def flush(qp, qn, rid, fc, nf):
            def fbody(j, fcj):
                for g in range(8):
                    stp[0, pl.ds(g * 16, 16)] = qp[pl.ds(j * 128 + g * 16, 16)]
                    stn[0, pl.ds(g * 16, 16)] = qn[pl.ds(j * 128 + g * 16, 16)]
                pltpu.sync_copy(stp, qpk_h.at[rid, fcj])
                pltpu.sync_copy(stn, qnm_h.at[rid, fcj])
                return fcj + 1

            fc = lax.fori_loop(0, nf, fbody, fc)
            for g in range(8):
                pv = qp[pl.ds(nf * 128 + g * 16, 16)]
                nv = qn[pl.ds(nf * 128 + g * 16, 16)]
                qp[pl.ds(g * 16, 16)] = pv
                qn[pl.ds(g * 16, 16)] = nv
            return fc

        def chunk(i, carry):
            qt0, fc0, qt1, fc1 = carry
            ro = i * 8
            pltpu.sync_copy(src_h.at[pl.ds(ro, 8)], srcbuf)
            pltpu.sync_copy(dst_h.at[pl.ds(ro, 8)], dstbuf)
            for r in range(8):
                for kk in range(8):
                    svec = srcbuf[r, pl.ds(kk * 16, 16)]
                    dvec = dstbuf[r, pl.ds(kk * 16, 16)]
                    dsg = plsc.load_gather(dinv_v, [svec])
                    ddg = plsc.load_gather(dinv_v, [jnp.minimum(dvec, NP - 1)])
                    nm = dsg * ddg
                    for (qp, qn, rid), which in ((queues[0], 0), (queues[1], 1)):
                        base = rid * _RT
                        m = (dvec >= base) & (dvec < base + _RT)
                        pk = svec * 1024 + (dvec - base)
                        qt = qt0 if which == 0 else qt1
                        plsc.store_compressed(qp.at[pl.ds(qt, 16)], pk, mask=m)
                        plsc.store_compressed(qn.at[pl.ds(qt, 16)], nm, mask=m)
                        cnt = jnp.sum(m.astype(i32))
                        if which == 0:
                            qt0 = qt + cnt
                        else:
                            qt1 = qt + cnt
            nf0 = qt0 // 128
            fc0 = flush(q0p, q0n, w, fc0, nf0)
            qt0 = qt0 - nf0 * 128
            nf1 = qt1 // 128
            fc1 = flush(q1p, q1n, 32 + w, fc1, nf1)
            qt1 = qt1 - nf1 * 128
            return qt0, fc0, qt1, fc1

        qt0, fc0, qt1, fc1 = lax.fori_loop(
            0, 800, chunk, (jnp.int32(0), jnp.int32(0), jnp.int32(0), jnp.int32(0)))

        sent_pk = jnp.full((16,), _SENT, i32)
        sent_nm = jnp.zeros((16,), f32)
        for (qp, qn, rid), qt, fc in ((queues[0], qt0, fc0), (queues[1], qt1, fc1)):
            for g in range(8):
                qp[pl.ds(qt + g * 16, 16)] = sent_pk
                qn[pl.ds(qt + g * 16, 16)] = sent_nm
            for g in range(8):
                stp[0, pl.ds(g * 16, 16)] = qp[pl.ds(g * 16, 16)]
                stn[0, pl.ds(g * 16, 16)] = qn[pl.ds(g * 16, 16)]
            pltpu.sync_copy(stp, qpk_h.at[rid, fc])
            pltpu.sync_copy(stn, qnm_h.at[rid, fc])
            total = fc + jnp.where(qt > 0, 1, 0).astype(i32)
            szb[0, pl.ds(0, 16)] = jnp.broadcast_to(total, (16,))
            pltpu.sync_copy(szb, qsz_h.at[rid])

    return k(src2d, dst2d, dinv1d)


def _sc_segsum(qpk, qnm, qsz, tab, dinv1d, zeros_h):
    @functools.partial(
        pl.kernel,
        out_type=jax.ShapeDtypeStruct((NP, EMB), f32),
        compiler_params=pltpu.CompilerParams(needs_layout_passes=False),
        mesh=_mesh(),
        scratch_types=[
            pltpu.VMEM((_ACC, EMB), f32),
            pltpu.VMEM((128, EMB), f32),
            pltpu.VMEM((1, 128), i32),
            pltpu.VMEM((1, 128), f32),
            pltpu.VMEM((128,), i32),
            pltpu.VMEM((_RT,), f32),
            pltpu.VMEM((1, 16), i32),
            pltpu.SemaphoreType.DMA,
        ],
    )
    def k(qpk_h, qnm_h, qsz_h, tab_h, dinv_h, z_h, out_h,
          acc, rowbuf, pkbuf, nmbuf, gidxv, dv_v, szv, sem):
        c = lax.axis_index("c")
        s = lax.axis_index("s")
        w = c * 16 + s
        it16 = _iota16()
        for p in range(2):
            rid = p * 32 + w
            base = rid * _RT
            pltpu.sync_copy(z_h, acc)
            pltpu.sync_copy(qsz_h.at[rid], szv)
            nch = jnp.max(szv[0, pl.ds(0, 16)])

            def chunk(j, _):
                pltpu.sync_copy(qpk_h.at[rid, j], pkbuf)
                pltpu.sync_copy(qnm_h.at[rid, j], nmbuf)
                for g in range(8):
                    pk = pkbuf[0, pl.ds(g * 16, 16)]
                    gidxv[pl.ds(g * 16, 16)] = pk // 1024
                pltpu.async_copy(tab_h.at[gidxv], rowbuf, sem).wait()
                offs = []
                nms = []
                for g in range(8):
                    pk = pkbuf[0, pl.ds(g * 16, 16)]
                    offs.append(pk - (pk // 1024) * 1024)
                    nms.append(nmbuf[0, pl.ds(g * 16, 16)])

                for g in range(8):
                    lanes = g * 16 + it16
                    off16 = offs[g]
                    nm16 = nms[g]

                    def gsweep(t, _2):
                        cvec = t - it16
                        m = (cvec >= 0) & (cvec < 128)
                        cc = jnp.bitwise_and(cvec, 127)
                        v = plsc.load_gather(rowbuf, [lanes, cc], mask=m)
                        tt = v * nm16
                        plsc.addupdate_scatter(acc, [off16, cc], tt, mask=m)
                        return 0

                    lax.fori_loop(0, 143, gsweep, 0)
                return 0

            lax.fori_loop(0, nch, chunk, 0)

            pltpu.sync_copy(dinv_h.at[pl.ds(base, _RT)], dv_v)
            for cc in range(7):
                pltpu.sync_copy(tab_h.at[pl.ds(base + cc * 112, 112)],
                                rowbuf.at[pl.ds(0, 112)])
                for g in range(7):
                    rowoff = cc * 112 + g * 16
                    lanes = g * 16 + it16
                    dv = dv_v[pl.ds(rowoff, 16)]
                    nn = dv * dv
                    offv = rowoff + it16

                    def slbody(col, _2):
                        colv = jnp.bitwise_and(col + it16, 127)
                        v = plsc.load_gather(rowbuf, [lanes, colv])
                        t = v * nn
                        plsc.addupdate_scatter(acc, [offv, colv], t)
                        return 0

                    lax.fori_loop(0, 128, slbody, 0)

            pltpu.sync_copy(acc.at[pl.ds(0, _RT)],
                            out_h.at[pl.ds(base, _RT)])

    return k(qpk, qnm, qsz, tab, dinv1d, zeros_h)


def _sc_gather(idx, x1, x2, x3, x4f):
    @functools.partial(
        pl.kernel,
        out_type=tuple(
            jax.ShapeDtypeStruct((1024, EMB), f32) for _ in range(4)),
        compiler_params=pltpu.CompilerParams(needs_layout_passes=False),
        mesh=_mesh(),
        scratch_types=[
            pltpu.VMEM((1, 32), i32),
            pltpu.VMEM((32, EMB), f32),
            pltpu.VMEM((32, EMB), f32),
            pltpu.VMEM((32, EMB), f32),
            pltpu.VMEM((32, EMB), f32),
            pltpu.SemaphoreType.DMA,
        ],
    )
    def k(idx_h, x1_h, x2_h, x3_h, x4_h, p1_h, p2_h, p3_h, p4_h,
          idxv, r1, r2, r3, r4, sem):
        c = lax.axis_index("c")
        s = lax.axis_index("s")
        w = c * 16 + s
        pltpu.sync_copy(idx_h.at[w], idxv)
        ds_ = [pltpu.async_copy(x1_h.at[idxv.at[0]], r1, sem),
               pltpu.async_copy(x2_h.at[idxv.at[0]], r2, sem),
               pltpu.async_copy(x3_h.at[idxv.at[0]], r3, sem),
               pltpu.async_copy(x4_h.at[idxv.at[0]], r4, sem)]
        for d in ds_:
            d.wait()
        pltpu.sync_copy(r1, p1_h.at[pl.ds(w * 32, 32)])
        pltpu.sync_copy(r2, p2_h.at[pl.ds(w * 32, 32)])
        pltpu.sync_copy(r3, p3_h.at[pl.ds(w * 32, 32)])
        pltpu.sync_copy(r4, p4_h.at[pl.ds(w * 32, 32)])

    return k(idx, x1, x2, x3, x4f)


_BN = 1024


def _tc_mm0(x, W1, deg):
    def body(x_ref, w_ref, dg_ref, dinv_ref, hw_ref):
        d = dg_ref[:, 0:1] + 1.0
        dinv_ref[...] = lax.rsqrt(d)
        hw_ref[...] = jnp.dot(x_ref[...], w_ref[...],
                              preferred_element_type=f32)

    return pl.pallas_call(
        body,
        grid=(NP // _BN,),
        in_specs=[
            pl.BlockSpec((_BN, F_IN), lambda i: (i, 0)),
            pl.BlockSpec((F_IN, EMB), lambda i: (0, 0)),
            pl.BlockSpec((_BN, EMB), lambda i: (i, 0)),
        ],
        out_specs=[
            pl.BlockSpec((_BN, 1), lambda i: (i, 0)),
            pl.BlockSpec((_BN, EMB), lambda i: (i, 0)),
        ],
        out_shape=[
            jax.ShapeDtypeStruct((NP, 1), f32),
            jax.ShapeDtypeStruct((NP, EMB), f32),
        ],
    )(x, W1, deg)


def _tc_layer(sa, b, Wn):
    def body(s_ref, b_ref, w_ref, x_ref, hwn_ref):
        t = jnp.tanh(s_ref[...] + b_ref[...])
        x_ref[...] = t
        hwn_ref[...] = jnp.dot(t, w_ref[...], preferred_element_type=f32)

    return pl.pallas_call(
        body,
        grid=(NP // _BN,),
        in_specs=[
            pl.BlockSpec((_BN, EMB), lambda i: (i, 0)),
            pl.BlockSpec((1, EMB), lambda i: (0, 0)),
            pl.BlockSpec((EMB, EMB), lambda i: (0, 0)),
        ],
        out_specs=[
            pl.BlockSpec((_BN, EMB), lambda i: (i, 0)),
            pl.BlockSpec((_BN, EMB), lambda i: (i, 0)),
        ],
        out_shape=[
            jax.ShapeDtypeStruct((NP, EMB), f32),
            jax.ShapeDtypeStruct((NP, EMB), f32),
        ],
    )(sa, b, Wn)


def _tc_layer4(s4, b4):
    def body(s_ref, b_ref, x_ref):
        x_ref[...] = jnp.tanh(s_ref[...] + b_ref[...])

    return pl.pallas_call(
        body,
        grid=(NP // _BN,),
        in_specs=[
            pl.BlockSpec((_BN, EMB), lambda i: (i, 0)),
            pl.BlockSpec((1, EMB), lambda i: (0, 0)),
        ],
        out_specs=pl.BlockSpec((_BN, EMB), lambda i: (i, 0)),
        out_shape=jax.ShapeDtypeStruct((NP, EMB), f32),
    )(s4, b4)


_NROW = 392


def _tc_topk(score2d, batch2d):
    def body(score_ref, batch_ref, idx_ref, maskc_ref):
        idx_ref[...] = jnp.zeros((32, 1, 32), i32)
        maskc_ref[...] = jnp.zeros((1024, 1), f32)
        score = score_ref[...]
        batchv = batch_ref[...]
        iota2 = (lax.broadcasted_iota(i32, (_NROW, 128), 0) * 128
                 + lax.broadcasted_iota(i32, (_NROW, 128), 1))
        lane = lax.broadcasted_iota(i32, (1, 32), 1)

        def gbody(g, _):
            sg0 = jnp.where(batchv == g, score, -jnp.inf)

            def kbody(kk, carry):
                sg, idxrow = carry
                v = jnp.max(sg)
                flat = jnp.where(sg == v, iota2, jnp.int32(2 ** 30))
                fi = jnp.min(flat)
                sg = jnp.where(iota2 == fi, -jnp.inf, sg)
                idxrow = jnp.where(lane == kk, jnp.minimum(fi, N - 1), idxrow)
                valid = (v > -jnp.inf).astype(f32)
                maskc_ref[pl.ds(g * 32 + kk, 1), :] = valid.reshape(1, 1)
                return sg, idxrow

            sg, idxrow = lax.fori_loop(0, K, kbody,
                                       (sg0, jnp.zeros((1, 32), i32)))
            idx_ref[pl.ds(g, 1)] = idxrow.reshape(1, 1, 32)
            return 0

        lax.fori_loop(0, G, gbody, 0)

    return pl.pallas_call(
        body,
        out_shape=[
            jax.ShapeDtypeStruct((32, 1, 32), i32),
            jax.ShapeDtypeStruct((1024, 1), f32),
        ],
    )(score2d, batch2d)


def _tc_head_a(p1, p2, p3, p4, maskc, W5p, b5, W6r, b6):
    def body(p1_ref, p2_ref, p3_ref, p4_ref, m_ref, w5_ref, b5_ref,
             w6_ref, b6_ref, y6_ref):
        m = m_ref[...]
        xc = jnp.concatenate(
            [p1_ref[...], p2_ref[...], p3_ref[...], p4_ref[:, :16]],
            axis=-1) * m
        y5 = jax.nn.relu(
            jnp.dot(xc, w5_ref[...], preferred_element_type=f32) + b5_ref[...])
        zp = jnp.max(y5.reshape(512, 2, 64), axis=1)
        zp3 = zp.reshape(32, 16, 64)
        u = jnp.concatenate([zp3[:, j:j + 11, :] for j in range(5)], axis=-1)
        y6 = jax.nn.relu(
            jnp.dot(u.reshape(352, 320), w6_ref[...],
                    preferred_element_type=f32) + b6_ref[...])
        y6_ref[...] = y6.reshape(32, 11, EMB)

    return pl.pallas_call(
        body,
        out_shape=jax.ShapeDtypeStruct((32, 11, EMB), f32),
    )(p1, p2, p3, p4, maskc, W5p, b5, W6r, b6)


def _tc_head_b(emb2, W1c, b1, W2c, b2):
    def body(e_ref, w1_ref, b1_ref, w2_ref, b2_ref, o_ref, acc_ref):
        i = pl.program_id(0)

        @pl.when(i == 0)
        def _():
            acc_ref[...] = jnp.zeros((1, 384), f32)

        acc_ref[...] += jnp.dot(e_ref[...], w1_ref[...],
                                preferred_element_type=f32)

        @pl.when(i == G - 1)
        def _():
            h = jnp.tanh(acc_ref[...] + b1_ref[...])
            o_ref[...] = jnp.tanh(
                jnp.dot(h, w2_ref[...], preferred_element_type=f32)
                + b2_ref[...])

    return pl.pallas_call(
        body,
        grid=(G,),
        in_specs=[
            pl.BlockSpec((1, DENSE), lambda i: (0, i)),
            pl.BlockSpec((DENSE, 384), lambda i: (i, 0)),
            pl.BlockSpec((1, 384), lambda i: (0, 0)),
            pl.BlockSpec((384, OUT), lambda i: (0, 0)),
            pl.BlockSpec((1, OUT), lambda i: (0, 0)),
        ],
        out_specs=pl.BlockSpec((1, OUT), lambda i: (0, 0)),
        out_shape=jax.ShapeDtypeStruct((1, OUT), f32),
        scratch_shapes=[pltpu.VMEM((1, 384), f32)],
    )(emb2, W1c, b1, W2c, b2)


def kernel(x, edge_index, batch, W1g, b1g, W2g, b2g, W3g, b3g, W4g, b4g,
           conv5_w, conv5_b, conv6_w, conv6_b, cls1_W, cls1_b, cls2_W, cls2_b):
    xpad = jnp.pad(x, ((0, NP - N), (0, 0)))
    epad = EP - E
    src2d = jnp.concatenate(
        [edge_index[0], jnp.zeros((epad,), i32)]).reshape(_EROWS, 128)
    dst2d = jnp.concatenate(
        [edge_index[1], jnp.full((epad,), NP, i32)]).reshape(_EROWS, 128)
    zeros_deg = jnp.zeros((792, EMB), f32)
    zeros_acc = jnp.zeros((_ACC, EMB), f32)
    ones_h = jnp.ones((128, EMB), f32)

    deg = _sc_deg(dst2d, zeros_deg, ones_h)
    dinv, hw1 = _tc_mm0(xpad, W1g, deg)
    dinv1d = dinv.reshape(NP)

    qpk, qnm, qsz = _sc_build(src2d, dst2d, dinv1d)

    s1 = _sc_segsum(qpk, qnm, qsz, hw1, dinv1d, zeros_acc)
    x1, hw2 = _tc_layer(s1, b1g.reshape(1, EMB), W2g)

    s2 = _sc_segsum(qpk, qnm, qsz, hw2, dinv1d, zeros_acc)
    x2, hw3 = _tc_layer(s2, b2g.reshape(1, EMB), W3g)

    s3 = _sc_segsum(qpk, qnm, qsz, hw3, dinv1d, zeros_acc)
    W4p = jnp.pad(W4g, ((0, 0), (0, EMB - 1)))
    x3, hw4 = _tc_layer(s3, b3g.reshape(1, EMB), W4p)

    s4 = _sc_segsum(qpk, qnm, qsz, hw4, dinv1d, zeros_acc)
    b4r = jnp.broadcast_to(b4g.reshape(1, 1), (1, EMB))
    x4f = _tc_layer4(s4, b4r)

    sflat = x4f[:N, 0]
    pad = NP - N
    score2d = jnp.concatenate(
        [sflat, jnp.full((pad,), -jnp.inf, f32)]).reshape(_NROW, 128)
    batch2d = jnp.concatenate(
        [batch, jnp.full((pad,), -1, i32)]).reshape(_NROW, 128)
    idx, maskc = _tc_topk(score2d, batch2d)

    p1, p2, p3, p4 = _sc_gather(idx, x1, x2, x3, x4f)

    W5p = jnp.pad(conv5_w.reshape(EMB // 2, 385).T, ((0, 15), (0, 0)))
    W6r = jnp.transpose(conv6_w, (2, 1, 0)).reshape(320, EMB)
    y6 = _tc_head_a(p1, p2, p3, p4, maskc, W5p, conv5_b.reshape(1, 64),
                    W6r, conv6_b.reshape(1, EMB))

    emb2 = jnp.transpose(y6[:G], (0, 2, 1)).reshape(1, G * DENSE)
    out = _tc_head_b(emb2, cls1_W, cls1_b.reshape(1, 384),
                     cls2_W, cls2_b.reshape(1, OUT))
    return out[0]
